# baseline (speedup 1.0000x reference)
import functools
import math

import jax
import jax.numpy as jnp
from jax import lax
from jax.experimental import pallas as pl
from jax.experimental.pallas import tpu as pltpu
from jax.experimental.pallas import tpu_sc as plsc

NC = 2
NS = 16
NW = NC * NS
LANES = 16


def _emb_body(nsteps, chunk, d, seq_len, scale,
              x_hbm, tt_hbm, idx_hbm, tok_hbm, pos_hbm, seg_hbm, out_hbm,
              ti_v, si_v, pi_v, tok_v, pos_v, seg_v, sem_t, sem_p, sem_s):
    wid = lax.axis_index("s") * NC + lax.axis_index("c")
    base = wid * (nsteps * chunk)
    l0 = lax.rem(base, seq_len)
    dchunks = d // LANES

    def step(c, carry):
        off = base + c * chunk
        loff = l0 + c * chunk
        pltpu.sync_copy(x_hbm.at[pl.ds(off, chunk)], ti_v)
        pltpu.sync_copy(tt_hbm.at[pl.ds(off, chunk)], si_v)
        pltpu.sync_copy(idx_hbm.at[pl.ds(loff, chunk)], pi_v)
        ct = pltpu.async_copy(tok_hbm.at[ti_v], tok_v, sem_t)
        cp = pltpu.async_copy(pos_hbm.at[pi_v], pos_v, sem_p)
        cs = pltpu.async_copy(seg_hbm.at[si_v], seg_v, sem_s)
        ct.wait()
        cp.wait()
        cs.wait()

        def row(t, rcarry):
            for j in range(dchunks):
                sl = pl.ds(j * LANES, LANES)
                tok_v[t, sl] = (tok_v[t, sl] + pos_v[t, sl] + seg_v[t, sl]) * scale
            return rcarry

        lax.fori_loop(0, chunk, row, None)
        pltpu.sync_copy(tok_v, out_hbm.at[pl.ds(off, chunk)])
        return carry

    lax.fori_loop(0, nsteps, step, None)


def kernel(x, token_types, index, token_emb, pos_emb, seg_emb):
    B, L = x.shape
    V, d = token_emb.shape
    n = B * L
    tokens_per_worker = n // NW
    chunk = 32
    nsteps = tokens_per_worker // chunk
    scale = math.sqrt(d)

    x_flat = x.reshape(-1).astype(jnp.int32)
    tt_flat = token_types.reshape(-1).astype(jnp.int32)
    idx = index.astype(jnp.int32)

    mesh = plsc.VectorSubcoreMesh(core_axis_name="c", subcore_axis_name="s")
    body = functools.partial(_emb_body, nsteps, chunk, d, L, scale)
    run = pl.kernel(
        body,
        mesh=mesh,
        out_type=jax.ShapeDtypeStruct((n, d), jnp.float32),
        scratch_types=[
            pltpu.VMEM((chunk,), jnp.int32),
            pltpu.VMEM((chunk,), jnp.int32),
            pltpu.VMEM((chunk,), jnp.int32),
            pltpu.VMEM((chunk, d), jnp.float32),
            pltpu.VMEM((chunk, d), jnp.float32),
            pltpu.VMEM((chunk, d), jnp.float32),
            pltpu.SemaphoreType.DMA,
            pltpu.SemaphoreType.DMA,
            pltpu.SemaphoreType.DMA,
        ],
    )
    out = run(x_flat, tt_flat, idx, token_emb, pos_emb, seg_emb)
    return out.reshape(B, L, d)


# speedup vs baseline: 2.2805x; 2.2805x over previous
import functools
import math

import jax
import jax.numpy as jnp
from jax import lax
from jax.experimental import pallas as pl
from jax.experimental.pallas import tpu as pltpu
from jax.experimental.pallas import tpu_sc as plsc

NC = 2
NS = 16
NW = NC * NS
LANES = 16
CHUNK = 32
_GDN = lax.GatherDimensionNumbers(
    offset_dims=(), collapsed_slice_dims=(0,), start_index_map=(0,))


def _lane(vec, lane):
    idx = jnp.full((LANES, 1), lane, dtype=jnp.int32)
    return lax.gather(vec, idx, _GDN, (1,),
                      mode=lax.GatherScatterMode.PROMISE_IN_BOUNDS)


def _emb_body(B, seq_len, d, scale,
              x_hbm, tt_hbm, idx_hbm, tok_hbm, pos_hbm, seg_hbm, out_hbm,
              pi0, pi1, pv0, pv1, seg_v, ti0, ti1, sv0, sv1,
              tb0, tb1, psem, gsem0, gsem1, osem0, osem1):
    wid = lax.axis_index("s") * NC + lax.axis_index("c")
    lpw = seq_len // NW
    l0 = wid * lpw
    dchunks = d // LANES
    ti = (ti0, ti1)
    sv = (sv0, sv1)
    tb = (tb0, tb1)
    gsem = (gsem0, gsem1)
    osem = (osem0, osem1)

    zoff = wid * 0
    pv = (pv0, pv1)
    pltpu.sync_copy(idx_hbm.at[pl.ds(l0, CHUNK)], pi0)
    pltpu.sync_copy(idx_hbm.at[pl.ds(l0 + CHUNK, CHUNK)], pi1)
    pc0 = pltpu.async_copy(pos_hbm.at[pi0], pv0, psem)
    pc1 = pltpu.async_copy(pos_hbm.at[pi1], pv1, psem)
    pltpu.sync_copy(seg_hbm.at[pl.ds(zoff, 3)], seg_v)

    iters = [(b, h) for b in range(B) for h in range(lpw // CHUNK)]
    n_it = len(iters)

    def fire(i, slot):
        b, h = iters[i]
        off = b * seq_len + l0 + h * CHUNK
        pltpu.sync_copy(x_hbm.at[pl.ds(off, CHUNK)], ti[slot])
        pltpu.sync_copy(tt_hbm.at[pl.ds(off, CHUNK)], sv[slot])
        return pltpu.async_copy(tok_hbm.at[ti[slot]], tb[slot], gsem[slot])

    gcopy = [None] * n_it
    ocopy = [None] * n_it
    gcopy[0] = fire(0, 0)
    pc0.wait()
    pc1.wait()

    for i in range(n_it):
        cur = i & 1
        b, h = iters[i]
        if i + 1 < n_it:
            gcopy[i + 1] = fire(i + 1, 1 - cur)
        gcopy[i].wait()

        buf = tb[cur]
        tts = sv[cur]
        pos_v = pv[h]

        def compute(j, carry, buf=buf, tts=tts, pos_v=pos_v):
            sl = pl.ds(j * LANES, LANES)
            s0 = seg_v[0, sl]
            s1 = seg_v[1, sl]
            s2 = seg_v[2, sl]
            d10 = s1 - s0
            d21 = s2 - s1

            def grp(g, gc):
                ttf = tts[pl.ds(g * LANES, LANES)].astype(jnp.float32)
                a16 = jnp.minimum(ttf, 1.0)
                b16 = jnp.maximum(ttf - 1.0, 0.0)
                for u in range(LANES):
                    t = g * LANES + u
                    seg_sel = s0 + _lane(a16, u) * d10 + _lane(b16, u) * d21
                    buf[t, sl] = (buf[t, sl] + pos_v[t, sl]
                                  + seg_sel) * scale
                return gc

            lax.fori_loop(0, CHUNK // LANES, grp, None)
            return carry

        lax.fori_loop(0, dchunks, compute, None)
        off = b * seq_len + l0 + h * CHUNK
        pltpu.sync_copy(buf, out_hbm.at[pl.ds(off, CHUNK)])


def kernel(x, token_types, index, token_emb, pos_emb, seg_emb):
    B, L = x.shape
    V, d = token_emb.shape
    n = B * L
    lpw = n // NW
    scale = math.sqrt(d)

    x_flat = x.reshape(-1).astype(jnp.int32)
    tt_flat = token_types.reshape(-1).astype(jnp.int32)
    idx = index.astype(jnp.int32)

    mesh = plsc.VectorSubcoreMesh(core_axis_name="c", subcore_axis_name="s")
    body = functools.partial(_emb_body, B, L, d, scale)
    run = pl.kernel(
        body,
        mesh=mesh,
        out_type=jax.ShapeDtypeStruct((n, d), jnp.float32),
        scratch_types=[
            pltpu.VMEM((CHUNK,), jnp.int32),
            pltpu.VMEM((CHUNK,), jnp.int32),
            pltpu.VMEM((CHUNK, d), jnp.float32),
            pltpu.VMEM((CHUNK, d), jnp.float32),
            pltpu.VMEM((3, d), jnp.float32),
            pltpu.VMEM((CHUNK,), jnp.int32),
            pltpu.VMEM((CHUNK,), jnp.int32),
            pltpu.VMEM((CHUNK,), jnp.int32),
            pltpu.VMEM((CHUNK,), jnp.int32),
            pltpu.VMEM((CHUNK, d), jnp.float32),
            pltpu.VMEM((CHUNK, d), jnp.float32),
            pltpu.SemaphoreType.DMA,
            pltpu.SemaphoreType.DMA,
            pltpu.SemaphoreType.DMA,
            pltpu.SemaphoreType.DMA,
            pltpu.SemaphoreType.DMA,
        ],
    )
    out = run(x_flat, tt_flat, idx, token_emb, pos_emb, seg_emb)
    return out.reshape(B, L, d)


# speedup vs baseline: 2.4441x; 1.0718x over previous
import functools
import math

import jax
import jax.numpy as jnp
from jax import lax
from jax.experimental import pallas as pl
from jax.experimental.pallas import tpu as pltpu
from jax.experimental.pallas import tpu_sc as plsc

NC = 2
NS = 16
NW = NC * NS
LANES = 16
CHUNK = 32
NBUF = 3
_GDN = lax.GatherDimensionNumbers(
    offset_dims=(), collapsed_slice_dims=(0,), start_index_map=(0,))


def _lane(vec, lane):
    idx = jnp.full((LANES, 1), lane, dtype=jnp.int32)
    return lax.gather(vec, idx, _GDN, (1,),
                      mode=lax.GatherScatterMode.PROMISE_IN_BOUNDS)


def _emb_body(B, seq_len, d, scale,
              x_hbm, tt_hbm, idx_hbm, tok_hbm, pos_hbm, seg_hbm, out_hbm,
              pi0, pi1, pv0, pv1, seg_v, ti0, ti1, ti2, sv0, sv1, sv2,
              tb0, tb1, tb2, psem, gsem0, gsem1, gsem2, osem0, osem1, osem2):
    wid = lax.axis_index("s") * NC + lax.axis_index("c")
    lpw = seq_len // NW
    l0 = wid * lpw
    dchunks = d // LANES
    ti = (ti0, ti1, ti2)
    sv = (sv0, sv1, sv2)
    tb = (tb0, tb1, tb2)
    gsem = (gsem0, gsem1, gsem2)
    osem = (osem0, osem1, osem2)

    zoff = wid * 0
    pv = (pv0, pv1)
    pltpu.sync_copy(idx_hbm.at[pl.ds(l0, CHUNK)], pi0)
    pltpu.sync_copy(idx_hbm.at[pl.ds(l0 + CHUNK, CHUNK)], pi1)
    pc0 = pltpu.async_copy(pos_hbm.at[pi0], pv0, psem)
    pc1 = pltpu.async_copy(pos_hbm.at[pi1], pv1, psem)
    pltpu.sync_copy(seg_hbm.at[pl.ds(zoff, 3)], seg_v)

    iters = [(b, h) for b in range(B) for h in range(lpw // CHUNK)]
    n_it = len(iters)

    def fire(i):
        b, h = iters[i]
        slot = i % NBUF
        off = b * seq_len + l0 + h * CHUNK
        pltpu.sync_copy(x_hbm.at[pl.ds(off, CHUNK)], ti[slot])
        pltpu.sync_copy(tt_hbm.at[pl.ds(off, CHUNK)], sv[slot])
        return pltpu.async_copy(tok_hbm.at[ti[slot]], tb[slot], gsem[slot])

    gcopy = [None] * n_it
    ocopy = [None] * n_it
    gcopy[0] = fire(0)
    gcopy[1] = fire(1)
    pc0.wait()
    pc1.wait()

    for i in range(n_it):
        cur = i % NBUF
        b, h = iters[i]
        gcopy[i].wait()

        buf = tb[cur]
        tts = sv[cur]
        pos_v = pv[h]

        def compute(j, carry, buf=buf, tts=tts, pos_v=pos_v):
            sl = pl.ds(j * LANES, LANES)
            s0 = seg_v[0, sl]
            s1 = seg_v[1, sl]
            s2 = seg_v[2, sl]
            d10 = s1 - s0
            d21 = s2 - s1

            def grp(g, gc):
                ttf = tts[pl.ds(g * LANES, LANES)].astype(jnp.float32)
                a16 = jnp.minimum(ttf, 1.0)
                b16 = jnp.maximum(ttf - 1.0, 0.0)
                for u in range(LANES):
                    t = g * LANES + u
                    seg_sel = s0 + _lane(a16, u) * d10 + _lane(b16, u) * d21
                    buf[t, sl] = (buf[t, sl] + pos_v[t, sl]
                                  + seg_sel) * scale
                return gc

            lax.fori_loop(0, CHUNK // LANES, grp, None)
            return carry

        lax.fori_loop(0, dchunks, compute, None)
        off = b * seq_len + l0 + h * CHUNK
        ocopy[i] = pltpu.async_copy(buf, out_hbm.at[pl.ds(off, CHUNK)],
                                    osem[cur])
        if i + 2 < n_it:
            if i >= 1:
                ocopy[i - 1].wait()
            gcopy[i + 2] = fire(i + 2)

    for i in range(max(0, n_it - 3), n_it):
        ocopy[i].wait()


def kernel(x, token_types, index, token_emb, pos_emb, seg_emb):
    B, L = x.shape
    V, d = token_emb.shape
    n = B * L
    scale = math.sqrt(d)

    x_flat = x.reshape(-1).astype(jnp.int32)
    tt_flat = token_types.reshape(-1).astype(jnp.int32)
    idx = index.astype(jnp.int32)

    mesh = plsc.VectorSubcoreMesh(core_axis_name="c", subcore_axis_name="s")
    body = functools.partial(_emb_body, B, L, d, scale)
    run = pl.kernel(
        body,
        mesh=mesh,
        out_type=jax.ShapeDtypeStruct((n, d), jnp.float32),
        scratch_types=[
            pltpu.VMEM((CHUNK,), jnp.int32),
            pltpu.VMEM((CHUNK,), jnp.int32),
            pltpu.VMEM((CHUNK, d), jnp.float32),
            pltpu.VMEM((CHUNK, d), jnp.float32),
            pltpu.VMEM((3, d), jnp.float32),
            pltpu.VMEM((CHUNK,), jnp.int32),
            pltpu.VMEM((CHUNK,), jnp.int32),
            pltpu.VMEM((CHUNK,), jnp.int32),
            pltpu.VMEM((CHUNK,), jnp.int32),
            pltpu.VMEM((CHUNK,), jnp.int32),
            pltpu.VMEM((CHUNK,), jnp.int32),
            pltpu.VMEM((CHUNK, d), jnp.float32),
            pltpu.VMEM((CHUNK, d), jnp.float32),
            pltpu.VMEM((CHUNK, d), jnp.float32),
            pltpu.SemaphoreType.DMA,
            pltpu.SemaphoreType.DMA,
            pltpu.SemaphoreType.DMA,
            pltpu.SemaphoreType.DMA,
            pltpu.SemaphoreType.DMA,
            pltpu.SemaphoreType.DMA,
            pltpu.SemaphoreType.DMA,
        ],
    )
    out = run(x_flat, tt_flat, idx, token_emb, pos_emb, seg_emb)
    return out.reshape(B, L, d)


# speedup vs baseline: 3.1875x; 1.3041x over previous
import functools
import math

import jax
import jax.numpy as jnp
from jax import lax
from jax.experimental import pallas as pl
from jax.experimental.pallas import tpu as pltpu
from jax.experimental.pallas import tpu_sc as plsc

NC = 2
NS = 16
NW = NC * NS
LANES = 16
CHUNK = 32
NBUF = 3
_GDN = lax.GatherDimensionNumbers(
    offset_dims=(), collapsed_slice_dims=(0,), start_index_map=(0,))


def _lane(vec, lane):
    idx = jnp.full((LANES, 1), lane, dtype=jnp.int32)
    return lax.gather(vec, idx, _GDN, (1,),
                      mode=lax.GatherScatterMode.PROMISE_IN_BOUNDS)


def _emb_body(B, seq_len, d, scale,
              x_hbm, tt_hbm, idx_hbm, tok_hbm, pos_hbm, seg_hbm, out_hbm,
              xi_all, tt_all, pi0, pi1, pv0, pv1, seg_v,
              tb0, tb1, tb2, ssem, psem, gsem0, gsem1, gsem2,
              osem0, osem1, osem2):
    wid = lax.axis_index("s") * NC + lax.axis_index("c")
    lpw = seq_len // NW
    l0 = wid * lpw
    dchunks = d // LANES
    tb = (tb0, tb1, tb2)
    gsem = (gsem0, gsem1, gsem2)
    osem = (osem0, osem1, osem2)
    pv = (pv0, pv1)

    zoff = wid * 0
    stage = []
    for b in range(B):
        stage.append(pltpu.async_copy(x_hbm.at[b, pl.ds(l0, lpw)],
                                      xi_all.at[b], ssem))
        stage.append(pltpu.async_copy(tt_hbm.at[b, pl.ds(l0, lpw)],
                                      tt_all.at[b], ssem))
    pltpu.sync_copy(idx_hbm.at[pl.ds(l0, CHUNK)], pi0)
    pltpu.sync_copy(idx_hbm.at[pl.ds(l0 + CHUNK, CHUNK)], pi1)
    pc0 = pltpu.async_copy(pos_hbm.at[pi0], pv0, psem)
    pc1 = pltpu.async_copy(pos_hbm.at[pi1], pv1, psem)
    pltpu.sync_copy(seg_hbm.at[pl.ds(zoff, 3)], seg_v)
    for c in stage:
        c.wait()

    iters = [(b, h) for b in range(B) for h in range(lpw // CHUNK)]
    n_it = len(iters)

    def fire(i):
        b, h = iters[i]
        idx_ref = xi_all.at[b, pl.ds(h * CHUNK, CHUNK)]
        return pltpu.async_copy(tok_hbm.at[idx_ref], tb[i % NBUF],
                                gsem[i % NBUF])

    gcopy = [None] * n_it
    ocopy = [None] * n_it
    gcopy[0] = fire(0)
    gcopy[1] = fire(1)
    pc0.wait()
    pc1.wait()

    for i in range(n_it):
        cur = i % NBUF
        b, h = iters[i]
        gcopy[i].wait()

        buf = tb[cur]
        pos_v = pv[h]

        def compute(j, carry, buf=buf, pos_v=pos_v, b=b, h=h):
            sl = pl.ds(j * LANES, LANES)
            s0 = seg_v[0, sl]
            s1 = seg_v[1, sl]
            s2 = seg_v[2, sl]
            d10 = s1 - s0
            d21 = s2 - s1
            for g in range(CHUNK // LANES):
                tts = tt_all[b, h * CHUNK + g * LANES:
                             h * CHUNK + (g + 1) * LANES]
                ttf = tts.astype(jnp.float32)
                a16 = jnp.minimum(ttf, 1.0)
                b16 = jnp.maximum(ttf - 1.0, 0.0)
                for u in range(LANES):
                    t = g * LANES + u
                    seg_sel = (s0 + _lane(a16, u) * d10
                               + _lane(b16, u) * d21)
                    buf[t, sl] = (buf[t, sl] + pos_v[t, sl]
                                  + seg_sel) * scale
            return carry

        lax.fori_loop(0, dchunks, compute, None)
        off = b * seq_len + l0 + h * CHUNK
        ocopy[i] = pltpu.async_copy(buf, out_hbm.at[pl.ds(off, CHUNK)],
                                    osem[cur])
        if i + 2 < n_it:
            if i >= 1:
                ocopy[i - 1].wait()
            gcopy[i + 2] = fire(i + 2)

    for i in range(max(0, n_it - 3), n_it):
        ocopy[i].wait()


def kernel(x, token_types, index, token_emb, pos_emb, seg_emb):
    B, L = x.shape
    V, d = token_emb.shape
    n = B * L
    lpw = L // NW
    scale = math.sqrt(d)

    x2d = x.astype(jnp.int32)
    tt2d = token_types.astype(jnp.int32)
    idx = index.astype(jnp.int32)

    mesh = plsc.VectorSubcoreMesh(core_axis_name="c", subcore_axis_name="s")
    body = functools.partial(_emb_body, B, L, d, scale)
    run = pl.kernel(
        body,
        mesh=mesh,
        out_type=jax.ShapeDtypeStruct((n, d), jnp.float32),
        scratch_types=[
            pltpu.VMEM((B, lpw), jnp.int32),
            pltpu.VMEM((B, lpw), jnp.int32),
            pltpu.VMEM((CHUNK,), jnp.int32),
            pltpu.VMEM((CHUNK,), jnp.int32),
            pltpu.VMEM((CHUNK, d), jnp.float32),
            pltpu.VMEM((CHUNK, d), jnp.float32),
            pltpu.VMEM((3, d), jnp.float32),
            pltpu.VMEM((CHUNK, d), jnp.float32),
            pltpu.VMEM((CHUNK, d), jnp.float32),
            pltpu.VMEM((CHUNK, d), jnp.float32),
            pltpu.SemaphoreType.DMA,
            pltpu.SemaphoreType.DMA,
            pltpu.SemaphoreType.DMA,
            pltpu.SemaphoreType.DMA,
            pltpu.SemaphoreType.DMA,
            pltpu.SemaphoreType.DMA,
            pltpu.SemaphoreType.DMA,
            pltpu.SemaphoreType.DMA,
        ],
    )
    out = run(x2d, tt2d, idx, token_emb, pos_emb, seg_emb)
    return out.reshape(B, L, d)


# speedup vs baseline: 3.2445x; 1.0179x over previous
import functools
import math

import jax
import jax.numpy as jnp
from jax import lax
from jax.experimental import pallas as pl
from jax.experimental.pallas import tpu as pltpu
from jax.experimental.pallas import tpu_sc as plsc

NC = 2
NS = 16
NW = NC * NS
LANES = 16
CHUNK = 32
NBUF = 2
_GDN = lax.GatherDimensionNumbers(
    offset_dims=(), collapsed_slice_dims=(0,), start_index_map=(0,))


def _lane(vec, lane):
    idx = jnp.full((LANES, 1), lane, dtype=jnp.int32)
    return lax.gather(vec, idx, _GDN, (1,),
                      mode=lax.GatherScatterMode.PROMISE_IN_BOUNDS)


def _emb_body(B, seq_len, d, scale,
              x_hbm, tt_hbm, idx_hbm, tok_hbm, pos_hbm, seg_hbm, out_hbm,
              xi_all, tt_all, pi0, pi1, pos_v, seg_v,
              tb0, tb1, ob0, ob1, ssem, psem, gsem0, gsem1,
              osem0, osem1):
    wid = lax.axis_index("s") * NC + lax.axis_index("c")
    lpw = seq_len // NW
    l0 = wid * lpw
    dchunks = d // LANES
    tb = (tb0, tb1)
    ob = (ob0, ob1)
    gsem = (gsem0, gsem1)
    osem = (osem0, osem1)
    nh = lpw // CHUNK

    zoff = wid * 0
    stage = []
    for b in range(B):
        stage.append(pltpu.async_copy(x_hbm.at[b, pl.ds(l0, lpw)],
                                      xi_all.at[b], ssem))
        stage.append(pltpu.async_copy(tt_hbm.at[b, pl.ds(l0, lpw)],
                                      tt_all.at[b], ssem))
    pltpu.sync_copy(idx_hbm.at[pl.ds(l0, CHUNK)], pi0)
    pltpu.sync_copy(idx_hbm.at[pl.ds(l0 + CHUNK, CHUNK)], pi1)
    pcopy = pltpu.async_copy(pos_hbm.at[pi0], pos_v, psem)
    pltpu.sync_copy(seg_hbm.at[pl.ds(zoff, 3)], seg_v)
    for c in stage:
        c.wait()

    iters = [(b, h) for h in range(nh) for b in range(B)]
    n_it = len(iters)

    def fire(i):
        b, h = iters[i]
        idx_ref = xi_all.at[b, pl.ds(h * CHUNK, CHUNK)]
        return pltpu.async_copy(tok_hbm.at[idx_ref], tb[i % NBUF],
                                gsem[i % NBUF])

    gcopy = [None] * n_it
    ocopy = [None] * n_it
    gcopy[0] = fire(0)
    pcopy.wait()

    for i in range(n_it):
        cur = i % NBUF
        b, h = iters[i]
        gcopy[i].wait()
        if i + 1 < n_it:
            gcopy[i + 1] = fire(i + 1)
        if i >= NBUF:
            ocopy[i - NBUF].wait()
        if i == B:
            pcopy.wait()

        buf = tb[cur]
        obuf = ob[cur]

        def compute(j, carry, buf=buf, obuf=obuf, b=b, h=h):
            sl = pl.ds(j * LANES, LANES)
            s0 = seg_v[0, sl]
            s1 = seg_v[1, sl]
            s2 = seg_v[2, sl]
            d10 = s1 - s0
            d21 = s2 - s1
            for g in range(CHUNK // LANES):
                tts = tt_all[b, h * CHUNK + g * LANES:
                             h * CHUNK + (g + 1) * LANES]
                ttf = tts.astype(jnp.float32)
                a16 = jnp.minimum(ttf, 1.0)
                b16 = jnp.maximum(ttf - 1.0, 0.0)
                for u in range(LANES):
                    t = g * LANES + u
                    seg_sel = (s0 + _lane(a16, u) * d10
                               + _lane(b16, u) * d21)
                    obuf[t, sl] = (buf[t, sl] + pos_v[t, sl]
                                   + seg_sel) * scale
            return carry

        lax.fori_loop(0, dchunks, compute, None)
        off = b * seq_len + l0 + h * CHUNK
        ocopy[i] = pltpu.async_copy(obuf, out_hbm.at[pl.ds(off, CHUNK)],
                                    osem[cur])
        if i == B - 1:
            pcopy = pltpu.async_copy(pos_hbm.at[pi1], pos_v, psem)

    for i in range(max(0, n_it - NBUF), n_it):
        ocopy[i].wait()


def kernel(x, token_types, index, token_emb, pos_emb, seg_emb):
    B, L = x.shape
    V, d = token_emb.shape
    n = B * L
    lpw = L // NW
    scale = math.sqrt(d)

    x2d = x.astype(jnp.int32)
    tt2d = token_types.astype(jnp.int32)
    idx = index.astype(jnp.int32)

    mesh = plsc.VectorSubcoreMesh(core_axis_name="c", subcore_axis_name="s")
    body = functools.partial(_emb_body, B, L, d, scale)
    run = pl.kernel(
        body,
        mesh=mesh,
        out_type=jax.ShapeDtypeStruct((n, d), jnp.float32),
        scratch_types=[
            pltpu.VMEM((B, lpw), jnp.int32),
            pltpu.VMEM((B, lpw), jnp.int32),
            pltpu.VMEM((CHUNK,), jnp.int32),
            pltpu.VMEM((CHUNK,), jnp.int32),
            pltpu.VMEM((CHUNK, d), jnp.float32),
            pltpu.VMEM((3, d), jnp.float32),
            pltpu.VMEM((CHUNK, d), jnp.float32),
            pltpu.VMEM((CHUNK, d), jnp.float32),
            pltpu.VMEM((CHUNK, d), jnp.float32),
            pltpu.VMEM((CHUNK, d), jnp.float32),
            pltpu.SemaphoreType.DMA,
            pltpu.SemaphoreType.DMA,
            pltpu.SemaphoreType.DMA,
            pltpu.SemaphoreType.DMA,
            pltpu.SemaphoreType.DMA,
            pltpu.SemaphoreType.DMA,
        ],
    )
    out = run(x2d, tt2d, idx, token_emb, pos_emb, seg_emb)
    return out.reshape(B, L, d)


# speedup vs baseline: 3.2525x; 1.0025x over previous
import functools
import math

import jax
import jax.numpy as jnp
from jax import lax
from jax.experimental import pallas as pl
from jax.experimental.pallas import tpu as pltpu
from jax.experimental.pallas import tpu_sc as plsc

NC = 2
NS = 16
NW = NC * NS
LANES = 16
CHUNK = 16
NBUF = 2
_GDN = lax.GatherDimensionNumbers(
    offset_dims=(), collapsed_slice_dims=(0,), start_index_map=(0,))


def _lane(vec, lane):
    idx = jnp.full((LANES, 1), lane, dtype=jnp.int32)
    return lax.gather(vec, idx, _GDN, (1,),
                      mode=lax.GatherScatterMode.PROMISE_IN_BOUNDS)


def _emb_body(B, seq_len, d, scale,
              x_hbm, tt_hbm, idx_hbm, tok_hbm, pos_hbm, seg_hbm, out_hbm,
              xi_all, tt_all, pi_all, pos_v, seg_v,
              tb0, tb1, ob0, ob1, ssem, psem, gsem0, gsem1,
              osem0, osem1):
    wid = lax.axis_index("s") * NC + lax.axis_index("c")
    lpw = seq_len // NW
    l0 = wid * lpw
    dchunks = d // LANES
    tb = (tb0, tb1)
    ob = (ob0, ob1)
    gsem = (gsem0, gsem1)
    osem = (osem0, osem1)
    nh = lpw // CHUNK

    zoff = wid * 0
    stage = []
    for b in range(B):
        stage.append(pltpu.async_copy(x_hbm.at[b, pl.ds(l0, lpw)],
                                      xi_all.at[b], ssem))
        stage.append(pltpu.async_copy(tt_hbm.at[b, pl.ds(l0, lpw)],
                                      tt_all.at[b], ssem))
    pltpu.sync_copy(idx_hbm.at[pl.ds(l0, lpw)], pi_all)
    pcopy = pltpu.async_copy(pos_hbm.at[pi_all.at[pl.ds(0, CHUNK)]],
                             pos_v, psem)
    pltpu.sync_copy(seg_hbm.at[pl.ds(zoff, 3)], seg_v)
    for c in stage:
        c.wait()

    iters = [(b, h) for h in range(nh) for b in range(B)]
    n_it = len(iters)

    def fire(i):
        b, h = iters[i]
        idx_ref = xi_all.at[b, pl.ds(h * CHUNK, CHUNK)]
        return pltpu.async_copy(tok_hbm.at[idx_ref], tb[i % NBUF],
                                gsem[i % NBUF])

    gcopy = [None] * n_it
    ocopy = [None] * n_it
    gcopy[0] = fire(0)
    pcopy.wait()

    for i in range(n_it):
        cur = i % NBUF
        b, h = iters[i]
        gcopy[i].wait()
        if i + 1 < n_it:
            gcopy[i + 1] = fire(i + 1)
        if i >= NBUF:
            ocopy[i - NBUF].wait()
        if i > 0 and i % B == 0:
            pcopy.wait()

        buf = tb[cur]
        obuf = ob[cur]

        @plsc.parallel_loop(0, dchunks)
        def compute(j, buf=buf, obuf=obuf, b=b, h=h):
            sl = pl.ds(j * LANES, LANES)
            s0 = seg_v[0, sl]
            s1 = seg_v[1, sl]
            s2 = seg_v[2, sl]
            d10 = s1 - s0
            d21 = s2 - s1
            tts = tt_all[b, h * CHUNK:h * CHUNK + LANES]
            ttf = tts.astype(jnp.float32)
            a16 = jnp.minimum(ttf, 1.0)
            b16 = jnp.maximum(ttf - 1.0, 0.0)
            for u in range(LANES):
                seg_sel = (s0 + _lane(a16, u) * d10
                           + _lane(b16, u) * d21)
                obuf[u, sl] = (buf[u, sl] + pos_v[u, sl]
                               + seg_sel) * scale

        off = b * seq_len + l0 + h * CHUNK
        ocopy[i] = pltpu.async_copy(obuf, out_hbm.at[pl.ds(off, CHUNK)],
                                    osem[cur])
        if i % B == B - 1 and i + 1 < n_it:
            hn = (i + 1) // B
            pcopy = pltpu.async_copy(
                pos_hbm.at[pi_all.at[pl.ds(hn * CHUNK, CHUNK)]],
                pos_v, psem)

    for i in range(max(0, n_it - NBUF), n_it):
        ocopy[i].wait()


def kernel(x, token_types, index, token_emb, pos_emb, seg_emb):
    B, L = x.shape
    V, d = token_emb.shape
    n = B * L
    lpw = L // NW
    scale = math.sqrt(d)

    x2d = x.astype(jnp.int32)
    tt2d = token_types.astype(jnp.int32)
    idx = index.astype(jnp.int32)

    mesh = plsc.VectorSubcoreMesh(core_axis_name="c", subcore_axis_name="s")
    body = functools.partial(_emb_body, B, L, d, scale)
    run = pl.kernel(
        body,
        mesh=mesh,
        out_type=jax.ShapeDtypeStruct((n, d), jnp.float32),
        scratch_types=[
            pltpu.VMEM((B, lpw), jnp.int32),
            pltpu.VMEM((B, lpw), jnp.int32),
            pltpu.VMEM((lpw,), jnp.int32),
            pltpu.VMEM((CHUNK, d), jnp.float32),
            pltpu.VMEM((3, d), jnp.float32),
            pltpu.VMEM((CHUNK, d), jnp.float32),
            pltpu.VMEM((CHUNK, d), jnp.float32),
            pltpu.VMEM((CHUNK, d), jnp.float32),
            pltpu.VMEM((CHUNK, d), jnp.float32),
            pltpu.SemaphoreType.DMA,
            pltpu.SemaphoreType.DMA,
            pltpu.SemaphoreType.DMA,
            pltpu.SemaphoreType.DMA,
            pltpu.SemaphoreType.DMA,
            pltpu.SemaphoreType.DMA,
        ],
    )
    out = run(x2d, tt2d, idx, token_emb, pos_emb, seg_emb)
    return out.reshape(B, L, d)
